# baseline (speedup 1.0000x reference)
import functools

import jax
import jax.numpy as jnp
from jax import lax
from jax.experimental import pallas as pl
from jax.experimental.pallas import tpu as pltpu
from jax.experimental.pallas import tpu_sc as plsc

N = 10000
E = 160000
D_IN = 128
H = 512

NC = 2
NS = 16
NW = NC * NS
L = 16

NP = 10240
NB = 16
BKT = 640
F = 128
B = 80
CB = 3
NV = 2 * NW
CAP = 2544
CAPL = 2560
R = 83440
SEG_PAD = 2 * B
ACC_W = (BKT + 1) * F

SPAN_LO = 5008
SPAN_HI = 4992
OFF_HI = 16 * SPAN_LO
HALF_LO = 2512
HALF_HI = 2496

_mesh = plsc.VectorSubcoreMesh(
    core_axis_name="c", subcore_axis_name="s", num_cores=NC, num_subcores=NS)


def _preprocess_body(ei, bsrc, bdst, counts, degp,
                     src_v, dst_v, csrc, cdst, deg_l, cnt_s, tmp, red, shdeg):
    c = lax.axis_index("c")
    s = lax.axis_index("s")
    w = c * NS + s
    islow = w < 16

    zf = jnp.zeros((L,), jnp.float32)
    zi = jnp.zeros((L,), jnp.int32)
    trash = jnp.full((L,), BKT, jnp.int32)
    iota = lax.iota(jnp.int32, L)
    ones = jnp.ones((L,), jnp.float32)

    def zero_deg(i, _):
        deg_l[pl.ds(i * L, L)] = zf
        return 0
    lax.fori_loop(0, NP // L, zero_deg, 0)

    for h in range(2):
        off = jnp.where(islow, w * SPAN_LO + h * HALF_LO,
                        OFF_HI + (w - 16) * SPAN_HI + h * HALF_HI)
        if h == 0:
            ngroups = jnp.where(islow, HALF_LO // L, HALF_HI // L)
        else:
            ngroups = jnp.int32(HALF_HI // L)

        pltpu.sync_copy(ei.at[pl.ds(off, HALF_HI)],
                        src_v.at[pl.ds(0, HALF_HI)])
        pltpu.sync_copy(ei.at[pl.ds(E + off, HALF_HI)],
                        dst_v.at[pl.ds(0, HALF_HI)])
        if h == 0:
            @pl.when(islow)
            def _():
                pltpu.sync_copy(ei.at[pl.ds(off + HALF_HI, L)],
                                src_v.at[pl.ds(HALF_HI, L)])
                pltpu.sync_copy(ei.at[pl.ds(E + off + HALF_HI, L)],
                                dst_v.at[pl.ds(HALF_HI, L)])

        def group(j, cnts):
            sl = pl.ds(j * L, L)
            s16 = src_v[sl]
            d16 = dst_v[sl]
            plsc.addupdate_scatter(deg_l, [d16], ones)
            out = []
            for k in range(NB):
                m = (d16 >= k * BKT) & (d16 < (k + 1) * BKT)
                ck = cnts[k]
                plsc.store_compressed(csrc.at[pl.ds(k * CAPL + ck, L)], s16,
                                      mask=m)
                plsc.store_compressed(cdst.at[pl.ds(k * CAPL + ck, L)],
                                      d16 - k * BKT, mask=m)
                pc = plsc.all_reduce_population_count(m)
                out.append(ck + pc[0])
            return tuple(out)

        z32 = jnp.int32(0)
        cnts = lax.fori_loop(0, ngroups, group, (z32,) * NB)

        v = w * 2 + h
        cv = jnp.zeros((L,), jnp.int32)
        for k in range(NB):
            ck = cnts[k]
            csrc[pl.ds(k * CAPL + ck, L)] = zi
            cdst[pl.ds(k * CAPL + ck, L)] = trash
            nbk = (ck + (L - 1)) // L
            cv = jnp.where(iota == k, nbk, cv)
            base = (v * NB + k) * CAP
            pltpu.sync_copy(csrc.at[pl.ds(k * CAPL, CAP)],
                            bsrc.at[pl.ds(base, CAP)])
            pltpu.sync_copy(cdst.at[pl.ds(k * CAPL, CAP)],
                            bdst.at[pl.ds(base, CAP)])
        cnt_s[...] = cv
        pltpu.sync_copy(cnt_s, counts.at[pl.ds(v * L, L)])

    pltpu.sync_copy(deg_l, shdeg.at[pl.ds(s * NP, NP)])
    plsc.subcore_barrier()
    seg = NP // NS

    def zero_red(i, _):
        red[pl.ds(i * L, L)] = zf
        return 0
    lax.fori_loop(0, seg // L, zero_red, 0)
    for t in range(NS):
        pltpu.sync_copy(shdeg.at[pl.ds(t * NP + s * seg, seg)], tmp)

        def addt(i, _):
            sl = pl.ds(i * L, L)
            red[sl] += tmp[sl]
            return 0
        lax.fori_loop(0, seg // L, addt, 0)
    pltpu.sync_copy(red, degp.at[pl.ds(c * NP + s * seg, seg)])


@functools.partial(
    pl.kernel,
    out_type=(
        jax.ShapeDtypeStruct((NV * NB * CAP,), jnp.int32),
        jax.ShapeDtypeStruct((NV * NB * CAP,), jnp.int32),
        jax.ShapeDtypeStruct((NV * L,), jnp.int32),
        jax.ShapeDtypeStruct((NC * NP,), jnp.float32),
    ),
    mesh=_mesh,
    compiler_params=pltpu.CompilerParams(needs_layout_passes=False),
    scratch_types=[
        pltpu.VMEM((HALF_LO,), jnp.int32),
        pltpu.VMEM((HALF_LO,), jnp.int32),
        pltpu.VMEM((NB * CAPL,), jnp.int32),
        pltpu.VMEM((NB * CAPL,), jnp.int32),
        pltpu.VMEM((NP,), jnp.float32),
        pltpu.VMEM((L,), jnp.int32),
        pltpu.VMEM((NP // NS,), jnp.float32),
        pltpu.VMEM((NP // NS,), jnp.float32),
        pltpu.VMEM_SHARED((NS * NP,), jnp.float32),
    ],
)
def _preprocess(*refs):
    _preprocess_body(*refs)


def _merge_body(bsrc, bdst, counts, ssrc, sdst, scnt, cbuf, tsrc, tdst, cnt_s):
    c = lax.axis_index("c")
    s = lax.axis_index("s")
    k = s
    iota = lax.iota(jnp.int32, L)
    segb = (c * NB + k) * R

    pltpu.sync_copy(counts.at[pl.ds(c * 2 * NS * L, 2 * NS * L)], cbuf)

    off = jnp.int32(0)
    for v in range(2 * NS):
        vec = cbuf[pl.ds(v * L, L)]
        cnt_e = jnp.sum(jnp.where(iota == k, vec, 0)) * L
        off = pl.multiple_of(off, L)
        rawb = ((c * 2 * NS + v) * NB + k) * CAP
        pltpu.sync_copy(bsrc.at[pl.ds(rawb, CAP)], tsrc)
        pltpu.sync_copy(tsrc, ssrc.at[pl.ds(segb + off, CAP)])
        pltpu.sync_copy(bdst.at[pl.ds(rawb, CAP)], tdst)
        pltpu.sync_copy(tdst, sdst.at[pl.ds(segb + off, CAP)])
        off = off + cnt_e

    off = pl.multiple_of(off, L)
    zi = jnp.zeros((L,), jnp.int32)
    trash = jnp.full((L,), BKT, jnp.int32)
    for g in range(SEG_PAD // L):
        tsrc[pl.ds(g * L, L)] = zi
        tdst[pl.ds(g * L, L)] = trash
    pltpu.sync_copy(tsrc.at[pl.ds(0, SEG_PAD)], ssrc.at[pl.ds(segb + off, SEG_PAD)])
    pltpu.sync_copy(tdst.at[pl.ds(0, SEG_PAD)], sdst.at[pl.ds(segb + off, SEG_PAD)])

    nb = ((off + SEG_PAD - 1) // SEG_PAD) * (SEG_PAD // B)
    cnt_s[...] = jnp.where(iota == 0, nb, 0)
    pltpu.sync_copy(cnt_s, scnt.at[pl.ds((c * NB + k) * L, L)])


@functools.partial(
    pl.kernel,
    out_type=(
        jax.ShapeDtypeStruct((NC * NB * R,), jnp.int32),
        jax.ShapeDtypeStruct((NC * NB * R,), jnp.int32),
        jax.ShapeDtypeStruct((NC * NB * L,), jnp.int32),
    ),
    mesh=_mesh,
    compiler_params=pltpu.CompilerParams(needs_layout_passes=False),
    scratch_types=[
        pltpu.VMEM((2 * NS * L,), jnp.int32),
        pltpu.VMEM((CAP,), jnp.int32),
        pltpu.VMEM((CAP,), jnp.int32),
        pltpu.VMEM((L,), jnp.int32),
    ],
)
def _merge(*refs):
    _merge_body(*refs)


def _make_spmm(hl, npart):
    nslab = hl // F
    ntasks = NB * nslab
    rounds = (ntasks * npart) // NW

    def body(y, ssrc, sdst, scnt, out, isrcc, idstc, rows_a, rows_b, cnt_s,
             acc, sem_a, sem_b):
        c = lax.axis_index("c")
        s = lax.axis_index("s")
        tid = c * NS + s
        iota = lax.iota(jnp.int32, L)
        zf = jnp.zeros((L,), jnp.float32)
        def round_body(r, _):
            if npart == 1:
                task = tid + r * NW
                part = jnp.int32(0)
                seg_lo, seg_n = jnp.int32(0), 2
            else:
                task = tid // npart
                part = lax.rem(tid, jnp.int32(npart))
                seg_lo, seg_n = part, 1
            k = task // nslab
            slab = lax.rem(task, jnp.int32(nslab))
            lo = k * BKT

            def zacc(i, _):
                acc[pl.ds(i * L, L)] = zf
                return 0
            lax.fori_loop(0, ACC_W // L, zacc, 0)

            def seg_body(c2, _):
                pltpu.sync_copy(scnt.at[pl.ds((c2 * NB + k) * L, L)], cnt_s)
                nb = jnp.sum(jnp.where(iota == 0, cnt_s[...], 0))
                segb = (c2 * NB + k) * R
                nch = (nb + CB - 1) // CB

                def issue(b, buf, sem):
                    pltpu.async_copy(y.at[isrcc.at[pl.ds(b * B, B)]], buf,
                                     sem)

                def drain(b, buf, sem):
                    pltpu.make_async_copy(y.at[isrcc.at[pl.ds(b * B, B)]],
                                          buf, sem).wait()

                def accb(b, buf):
                    for jg in range(B // L):
                        off16 = idstc[pl.ds(b * B + jg * L, L)] * F
                        for l in range(L):
                            dof = off16[l]
                            e = jg * L + l
                            for cc in range(F // L):
                                sl = pl.ds(dof + cc * L, L)
                                acc[sl] += buf[e, pl.ds(cc * L, L)]

                def chunk(ch, _):
                    cb0 = ch * CB
                    m = jnp.minimum(jnp.int32(CB), nb - cb0)
                    ebase = segb + cb0 * B
                    pltpu.sync_copy(ssrc.at[pl.ds(ebase, CB * B)], isrcc)
                    pltpu.sync_copy(sdst.at[pl.ds(ebase, CB * B)], idstc)
                    for g in range(CB * B // L):
                        sl = pl.ds(g * L, L)
                        isrcc[sl] = isrcc[sl] + slab * NP
                    issue(0, rows_a, sem_a)
                    for b in range(CB):
                        if b % 2 == 0:
                            cur, csem, nxt, nsem = rows_a, sem_a, rows_b, sem_b
                        else:
                            cur, csem, nxt, nsem = rows_b, sem_b, rows_a, sem_a

                        @pl.when(b < m)
                        def _(b=b, cur=cur, csem=csem, nxt=nxt, nsem=nsem):
                            if b + 1 < CB:
                                @pl.when(b + 1 < m)
                                def _():
                                    issue(b + 1, nxt, nsem)
                            drain(b, cur, csem)
                            accb(b, cur)
                    return 0
                lax.fori_loop(0, nch, chunk, 0)
                return 0
            lax.fori_loop(seg_lo, seg_lo + seg_n, seg_body, 0)

            obase = ((part * nslab + slab) * NP + lo) * F
            pltpu.sync_copy(acc.at[pl.ds(0, BKT * F)],
                            out.at[pl.ds(obase, BKT * F)])
            return 0
        lax.fori_loop(0, rounds, round_body, 0)

    return pl.kernel(
        body,
        out_type=jax.ShapeDtypeStruct((npart * nslab * NP * F,), jnp.float32),
        mesh=_mesh,
        compiler_params=pltpu.CompilerParams(needs_layout_passes=False),
        scratch_types=[
            pltpu.VMEM((CB * B,), jnp.int32),
            pltpu.VMEM((CB * B,), jnp.int32),
            pltpu.VMEM((B, F), jnp.float32),
            pltpu.VMEM((B, F), jnp.float32),
            pltpu.VMEM((L,), jnp.int32),
            pltpu.VMEM((ACC_W,), jnp.float32),
            pltpu.SemaphoreType.DMA,
            pltpu.SemaphoreType.DMA,
        ],
    )


_spmm_128 = _make_spmm(D_IN, 2)
_spmm_512 = _make_spmm(H, 1)

NSL_D = D_IN // F
NSL_H = H // F


_BLK = 512
_GRID = NP // _BLK


def _pre_body(dp_ref, x_ref, dinv_ref, y0_ref):
    deg = jnp.sum(dp_ref[...], axis=1, keepdims=True) + 1.0
    dinv = lax.rsqrt(deg)
    dinv_ref[...] = dinv
    y0_ref[0, :, :] = x_ref[...] * dinv


def _k_pre(deg2, x_pad):
    return pl.pallas_call(
        _pre_body,
        grid=(_GRID,),
        in_specs=[
            pl.BlockSpec((_BLK, 2), lambda i: (i, 0)),
            pl.BlockSpec((_BLK, D_IN), lambda i: (i, 0)),
        ],
        out_specs=[
            pl.BlockSpec((_BLK, 1), lambda i: (i, 0)),
            pl.BlockSpec((NSL_D, _BLK, F), lambda i: (0, i, 0)),
        ],
        out_shape=[
            jax.ShapeDtypeStruct((NP, 1), jnp.float32),
            jax.ShapeDtypeStruct((NSL_D, NP, F), jnp.float32),
        ],
    )(deg2, x_pad)


def _layer0_body(sp_ref, y0_ref, dinv_ref, w_ref, b_ref, y1_ref):
    dinv = dinv_ref[...]
    t = (sp_ref[0, 0] + sp_ref[1, 0] + y0_ref[0]) * dinv
    h = jnp.dot(t, w_ref[...], preferred_element_type=jnp.float32) + b_ref[...]
    h = jnp.maximum(h, 0.0) * dinv
    for sl in range(NSL_H):
        y1_ref[sl, :, :] = h[:, sl * F:(sl + 1) * F]


def _k_layer0(S0p, y0, dinv, W0, b0):
    return pl.pallas_call(
        _layer0_body,
        grid=(_GRID,),
        in_specs=[
            pl.BlockSpec((2, NSL_D, _BLK, F), lambda i: (0, 0, i, 0)),
            pl.BlockSpec((NSL_D, _BLK, F), lambda i: (0, i, 0)),
            pl.BlockSpec((_BLK, 1), lambda i: (i, 0)),
            pl.BlockSpec((D_IN, H), lambda i: (0, 0)),
            pl.BlockSpec((1, H), lambda i: (0, 0)),
        ],
        out_specs=pl.BlockSpec((NSL_H, _BLK, F), lambda i: (0, i, 0)),
        out_shape=jax.ShapeDtypeStruct((NSL_H, NP, F), jnp.float32),
    )(S0p, y0, dinv, W0, b0)


def _layer1_body(s_ref, yp_ref, dinv_ref, w_ref, b_ref, y_ref):
    dinv = dinv_ref[...]
    h = b_ref[...]
    for sl in range(NSL_H):
        t = (s_ref[sl] + yp_ref[sl]) * dinv
        h = h + jnp.dot(t, w_ref[pl.ds(sl * F, F), :],
                        preferred_element_type=jnp.float32)
    h = jnp.maximum(h, 0.0) * dinv
    for sl in range(NSL_H):
        y_ref[sl, :, :] = h[:, sl * F:(sl + 1) * F]


def _k_layer1(S, yp, dinv, W, b):
    return pl.pallas_call(
        _layer1_body,
        grid=(_GRID,),
        in_specs=[
            pl.BlockSpec((NSL_H, _BLK, F), lambda i: (0, i, 0)),
            pl.BlockSpec((NSL_H, _BLK, F), lambda i: (0, i, 0)),
            pl.BlockSpec((_BLK, 1), lambda i: (i, 0)),
            pl.BlockSpec((H, H), lambda i: (0, 0)),
            pl.BlockSpec((1, H), lambda i: (0, 0)),
        ],
        out_specs=pl.BlockSpec((NSL_H, _BLK, F), lambda i: (0, i, 0)),
        out_shape=jax.ShapeDtypeStruct((NSL_H, NP, F), jnp.float32),
    )(S, yp, dinv, W, b)


def _final_body(s_ref, yp_ref, dinv_ref, w2_ref, b2_ref, wfc_ref, bfc_ref,
                o_ref):
    dinv = dinv_ref[...]
    h = b2_ref[...]
    for sl in range(NSL_H):
        t = (s_ref[sl] + yp_ref[sl]) * dinv
        h = h + jnp.dot(t, w2_ref[pl.ds(sl * F, F), :],
                        preferred_element_type=jnp.float32)
    h = jnp.maximum(h, 0.0)
    o_ref[...] = jnp.dot(
        h, wfc_ref[...], preferred_element_type=jnp.float32) + bfc_ref[...]


def _k_final(S, yp, dinv, W2, b2, Wfc, bfc):
    return pl.pallas_call(
        _final_body,
        grid=(_GRID,),
        in_specs=[
            pl.BlockSpec((NSL_H, _BLK, F), lambda i: (0, i, 0)),
            pl.BlockSpec((NSL_H, _BLK, F), lambda i: (0, i, 0)),
            pl.BlockSpec((_BLK, 1), lambda i: (i, 0)),
            pl.BlockSpec((H, H), lambda i: (0, 0)),
            pl.BlockSpec((1, H), lambda i: (0, 0)),
            pl.BlockSpec((H, D_IN), lambda i: (0, 0)),
            pl.BlockSpec((1, D_IN), lambda i: (0, 0)),
        ],
        out_specs=pl.BlockSpec((_BLK, D_IN), lambda i: (i, 0)),
        out_shape=jax.ShapeDtypeStruct((NP, D_IN), jnp.float32),
    )(S, yp, dinv, W2, b2, Wfc, bfc)


def kernel(x, edge_index, W0, b0, W1, b1, W2, b2, Wfc, bfc):
    ei_flat = edge_index.reshape(-1).astype(jnp.int32)
    x_pad = jnp.pad(x, ((0, NP - N), (0, 0)))

    bsrc, bdst, counts, degp = _preprocess(ei_flat)
    ssrc, sdst, scnt = _merge(bsrc, bdst, counts)
    deg2 = degp.reshape(NC, NP).T

    dinv, y0 = _k_pre(deg2, x_pad)
    S0 = _spmm_128(y0.reshape(NSL_D * NP, F), ssrc, sdst, scnt)
    y1 = _k_layer0(S0.reshape(2, NSL_D, NP, F), y0, dinv, W0,
                   b0.reshape(1, H))
    S1 = _spmm_512(y1.reshape(NSL_H * NP, F), ssrc, sdst, scnt)
    y2 = _k_layer1(S1.reshape(NSL_H, NP, F), y1, dinv, W1, b1.reshape(1, H))
    S2 = _spmm_512(y2.reshape(NSL_H * NP, F), ssrc, sdst, scnt)
    out = _k_final(S2.reshape(NSL_H, NP, F), y2, dinv, W2, b2.reshape(1, H),
                   Wfc, bfc.reshape(1, D_IN))
    return out[:N]


# speedup vs baseline: 1.2929x; 1.2929x over previous
import functools

import jax
import jax.numpy as jnp
from jax import lax
from jax.experimental import pallas as pl
from jax.experimental.pallas import tpu as pltpu
from jax.experimental.pallas import tpu_sc as plsc

N = 10000
E = 160000
D_IN = 128
H = 512

NC = 2
NS = 16
NW = NC * NS
L = 16

NP = 10240
NB = 16
BKT = 640
F = 128
B = 80
CB = 2
NV = 2 * NW
CAP = 2544
CAPL = 2560
R = 83440
SEG_PAD = 2 * B
ACC_W = (BKT + 1) * F

SPAN_LO = 5008
SPAN_HI = 4992
OFF_HI = 16 * SPAN_LO
HALF_LO = 2512
HALF_HI = 2496

_mesh = plsc.VectorSubcoreMesh(
    core_axis_name="c", subcore_axis_name="s", num_cores=NC, num_subcores=NS)


def _preprocess_body(ei, bsrc, bdst, counts, degp,
                     src_v, dst_v, csrc, cdst, deg_l, cnt_s, tmp, red, shdeg):
    c = lax.axis_index("c")
    s = lax.axis_index("s")
    w = c * NS + s
    islow = w < 16

    zf = jnp.zeros((L,), jnp.float32)
    zi = jnp.zeros((L,), jnp.int32)
    trash = jnp.full((L,), BKT, jnp.int32)
    iota = lax.iota(jnp.int32, L)
    ones = jnp.ones((L,), jnp.float32)

    def zero_deg(i, _):
        deg_l[pl.ds(i * L, L)] = zf
        return 0
    lax.fori_loop(0, NP // L, zero_deg, 0)

    for h in range(2):
        off = jnp.where(islow, w * SPAN_LO + h * HALF_LO,
                        OFF_HI + (w - 16) * SPAN_HI + h * HALF_HI)
        if h == 0:
            ngroups = jnp.where(islow, HALF_LO // L, HALF_HI // L)
        else:
            ngroups = jnp.int32(HALF_HI // L)

        pltpu.sync_copy(ei.at[pl.ds(off, HALF_HI)],
                        src_v.at[pl.ds(0, HALF_HI)])
        pltpu.sync_copy(ei.at[pl.ds(E + off, HALF_HI)],
                        dst_v.at[pl.ds(0, HALF_HI)])
        if h == 0:
            @pl.when(islow)
            def _():
                pltpu.sync_copy(ei.at[pl.ds(off + HALF_HI, L)],
                                src_v.at[pl.ds(HALF_HI, L)])
                pltpu.sync_copy(ei.at[pl.ds(E + off + HALF_HI, L)],
                                dst_v.at[pl.ds(HALF_HI, L)])

        def group(j, cnts):
            sl = pl.ds(j * L, L)
            s16 = src_v[sl]
            d16 = dst_v[sl]
            plsc.addupdate_scatter(deg_l, [d16], ones)
            out = []
            for k in range(NB):
                m = (d16 >= k * BKT) & (d16 < (k + 1) * BKT)
                ck = cnts[k]
                plsc.store_compressed(csrc.at[pl.ds(k * CAPL + ck, L)], s16,
                                      mask=m)
                plsc.store_compressed(cdst.at[pl.ds(k * CAPL + ck, L)],
                                      d16 - k * BKT, mask=m)
                pc = plsc.all_reduce_population_count(m)
                out.append(ck + pc[0])
            return tuple(out)

        z32 = jnp.int32(0)
        cnts = lax.fori_loop(0, ngroups, group, (z32,) * NB)

        v = w * 2 + h
        cv = jnp.zeros((L,), jnp.int32)
        for k in range(NB):
            ck = cnts[k]
            csrc[pl.ds(k * CAPL + ck, L)] = zi
            cdst[pl.ds(k * CAPL + ck, L)] = trash
            nbk = (ck + (L - 1)) // L
            cv = jnp.where(iota == k, nbk, cv)
            base = (v * NB + k) * CAP
            pltpu.sync_copy(csrc.at[pl.ds(k * CAPL, CAP)],
                            bsrc.at[pl.ds(base, CAP)])
            pltpu.sync_copy(cdst.at[pl.ds(k * CAPL, CAP)],
                            bdst.at[pl.ds(base, CAP)])
        cnt_s[...] = cv
        pltpu.sync_copy(cnt_s, counts.at[pl.ds(v * L, L)])

    pltpu.sync_copy(deg_l, shdeg.at[pl.ds(s * NP, NP)])
    plsc.subcore_barrier()
    seg = NP // NS

    def zero_red(i, _):
        red[pl.ds(i * L, L)] = zf
        return 0
    lax.fori_loop(0, seg // L, zero_red, 0)
    for t in range(NS):
        pltpu.sync_copy(shdeg.at[pl.ds(t * NP + s * seg, seg)], tmp)

        def addt(i, _):
            sl = pl.ds(i * L, L)
            red[sl] += tmp[sl]
            return 0
        lax.fori_loop(0, seg // L, addt, 0)
    pltpu.sync_copy(red, degp.at[pl.ds(c * NP + s * seg, seg)])


@functools.partial(
    pl.kernel,
    out_type=(
        jax.ShapeDtypeStruct((NV * NB * CAP,), jnp.int32),
        jax.ShapeDtypeStruct((NV * NB * CAP,), jnp.int32),
        jax.ShapeDtypeStruct((NV * L,), jnp.int32),
        jax.ShapeDtypeStruct((NC * NP,), jnp.float32),
    ),
    mesh=_mesh,
    compiler_params=pltpu.CompilerParams(needs_layout_passes=False),
    scratch_types=[
        pltpu.VMEM((HALF_LO,), jnp.int32),
        pltpu.VMEM((HALF_LO,), jnp.int32),
        pltpu.VMEM((NB * CAPL,), jnp.int32),
        pltpu.VMEM((NB * CAPL,), jnp.int32),
        pltpu.VMEM((NP,), jnp.float32),
        pltpu.VMEM((L,), jnp.int32),
        pltpu.VMEM((NP // NS,), jnp.float32),
        pltpu.VMEM((NP // NS,), jnp.float32),
        pltpu.VMEM_SHARED((NS * NP,), jnp.float32),
    ],
)
def _preprocess(*refs):
    _preprocess_body(*refs)


def _merge_body(bsrc, bdst, counts, ssrc, sdst, scnt, cbuf, tsrc, tdst, cnt_s):
    c = lax.axis_index("c")
    s = lax.axis_index("s")
    k = s
    iota = lax.iota(jnp.int32, L)
    segb = (c * NB + k) * R

    pltpu.sync_copy(counts.at[pl.ds(c * 2 * NS * L, 2 * NS * L)], cbuf)

    off = jnp.int32(0)
    for v in range(2 * NS):
        vec = cbuf[pl.ds(v * L, L)]
        cnt_e = jnp.sum(jnp.where(iota == k, vec, 0)) * L
        off = pl.multiple_of(off, L)
        rawb = ((c * 2 * NS + v) * NB + k) * CAP
        pltpu.sync_copy(bsrc.at[pl.ds(rawb, CAP)], tsrc)
        pltpu.sync_copy(tsrc, ssrc.at[pl.ds(segb + off, CAP)])
        pltpu.sync_copy(bdst.at[pl.ds(rawb, CAP)], tdst)
        pltpu.sync_copy(tdst, sdst.at[pl.ds(segb + off, CAP)])
        off = off + cnt_e

    off = pl.multiple_of(off, L)
    zi = jnp.zeros((L,), jnp.int32)
    trash = jnp.full((L,), BKT, jnp.int32)
    for g in range(SEG_PAD // L):
        tsrc[pl.ds(g * L, L)] = zi
        tdst[pl.ds(g * L, L)] = trash
    pltpu.sync_copy(tsrc.at[pl.ds(0, SEG_PAD)], ssrc.at[pl.ds(segb + off, SEG_PAD)])
    pltpu.sync_copy(tdst.at[pl.ds(0, SEG_PAD)], sdst.at[pl.ds(segb + off, SEG_PAD)])

    nb = ((off + SEG_PAD - 1) // SEG_PAD) * (SEG_PAD // B)
    cnt_s[...] = jnp.where(iota == 0, nb, 0)
    pltpu.sync_copy(cnt_s, scnt.at[pl.ds((c * NB + k) * L, L)])


@functools.partial(
    pl.kernel,
    out_type=(
        jax.ShapeDtypeStruct((NC * NB * R,), jnp.int32),
        jax.ShapeDtypeStruct((NC * NB * R,), jnp.int32),
        jax.ShapeDtypeStruct((NC * NB * L,), jnp.int32),
    ),
    mesh=_mesh,
    compiler_params=pltpu.CompilerParams(needs_layout_passes=False),
    scratch_types=[
        pltpu.VMEM((2 * NS * L,), jnp.int32),
        pltpu.VMEM((CAP,), jnp.int32),
        pltpu.VMEM((CAP,), jnp.int32),
        pltpu.VMEM((L,), jnp.int32),
    ],
)
def _merge(*refs):
    _merge_body(*refs)


def _make_spmm(hl, npart):
    nslab = hl // F
    ntasks = NB * nslab
    rounds = (ntasks * npart) // NW

    def body(y, ssrc, sdst, scnt, out, ia_s, ia_d, ib_s, ib_d,
             r0, r1, r2, r3, cnt_s, acc, ta, tb, s0, s1, s2, s3):
        rows = [r0, r1, r2, r3]
        sems = [s0, s1, s2, s3]
        c = lax.axis_index("c")
        s = lax.axis_index("s")
        tid = c * NS + s
        iota = lax.iota(jnp.int32, L)
        zf = jnp.zeros((L,), jnp.float32)
        def round_body(r, _):
            if npart == 1:
                task = tid + r * NW
                part = jnp.int32(0)
                seg_lo, seg_n = jnp.int32(0), 2
            else:
                task = tid // npart
                part = lax.rem(tid, jnp.int32(npart))
                seg_lo, seg_n = part, 1
            k = task // nslab
            slab = lax.rem(task, jnp.int32(nslab))
            lo = k * BKT

            def zacc(i, _):
                acc[pl.ds(i * L, L)] = zf
                return 0
            lax.fori_loop(0, ACC_W // L, zacc, 0)

            def seg_body(c2, _):
                pltpu.sync_copy(scnt.at[pl.ds((c2 * NB + k) * L, L)], cnt_s)
                nb = jnp.sum(jnp.where(iota == 0, cnt_s[...], 0))
                segb = (c2 * NB + k) * R
                nch = (nb + CB - 1) // CB

                def issue_idx(ch, isr, idt, sem):
                    ebase = segb + ch * (CB * B)
                    pltpu.async_copy(ssrc.at[pl.ds(ebase, CB * B)], isr, sem)
                    pltpu.async_copy(sdst.at[pl.ds(ebase, CB * B)], idt, sem)

                def wait_idx(isr, idt, sem):
                    pltpu.make_async_copy(ssrc.at[pl.ds(segb, CB * B)], isr,
                                          sem).wait()
                    pltpu.make_async_copy(sdst.at[pl.ds(segb, CB * B)], idt,
                                          sem).wait()

                def accb(b, buf, idt):
                    for jg in range(B // L):
                        off16 = idt[pl.ds(b * B + jg * L, L)] * F
                        for l in range(L):
                            dof = off16[l]
                            e = jg * L + l
                            for cc in range(F // L):
                                t = buf[e, pl.ds(cc * L, L)]
                                plsc.addupdate(
                                    acc.at[pl.ds(dof + cc * L, L)], t)

                def adj_fire(isr, rp, sp):
                    for g in range(CB * B // L):
                        sl = pl.ds(g * L, L)
                        isr[sl] = isr[sl] + slab * NP
                    for b in range(CB):
                        pltpu.async_copy(
                            y.at[isr.at[pl.ds(b * B, B)]], rows[rp + b],
                            sems[sp + b])

                def drain_acc(isr, idt, rp, sp):
                    for b in range(CB):
                        pltpu.make_async_copy(
                            y.at[isr.at[pl.ds(b * B, B)]], rows[rp + b],
                            sems[sp + b]).wait()
                        accb(b, rows[rp + b], idt)

                @pl.when(nch > 0)
                def _():
                    issue_idx(0, ia_s, ia_d, ta)

                @pl.when(nch > 1)
                def _():
                    issue_idx(1, ib_s, ib_d, tb)

                def pair(t, _):
                    ch_a = 2 * t
                    wait_idx(ia_s, ia_d, ta)
                    adj_fire(ia_s, 0, 0)

                    @pl.when(ch_a + 1 < nch)
                    def _():
                        wait_idx(ib_s, ib_d, tb)
                        adj_fire(ib_s, 2, 2)

                    drain_acc(ia_s, ia_d, 0, 0)

                    @pl.when(ch_a + 2 < nch)
                    def _():
                        issue_idx(ch_a + 2, ia_s, ia_d, ta)

                    @pl.when(ch_a + 1 < nch)
                    def _():
                        drain_acc(ib_s, ib_d, 2, 2)

                        @pl.when(ch_a + 3 < nch)
                        def _():
                            issue_idx(ch_a + 3, ib_s, ib_d, tb)
                    return 0
                lax.fori_loop(0, (nch + 1) // 2, pair, 0)
                return 0
            lax.fori_loop(seg_lo, seg_lo + seg_n, seg_body, 0)

            obase = ((part * nslab + slab) * NP + lo) * F
            pltpu.sync_copy(acc.at[pl.ds(0, BKT * F)],
                            out.at[pl.ds(obase, BKT * F)])
            return 0
        lax.fori_loop(0, rounds, round_body, 0)

    return pl.kernel(
        body,
        out_type=jax.ShapeDtypeStruct((npart * nslab * NP * F,), jnp.float32),
        mesh=_mesh,
        compiler_params=pltpu.CompilerParams(needs_layout_passes=False),
        scratch_types=[
            pltpu.VMEM((CB * B,), jnp.int32),
            pltpu.VMEM((CB * B,), jnp.int32),
            pltpu.VMEM((CB * B,), jnp.int32),
            pltpu.VMEM((CB * B,), jnp.int32),
            pltpu.VMEM((B, F), jnp.float32),
            pltpu.VMEM((B, F), jnp.float32),
            pltpu.VMEM((B, F), jnp.float32),
            pltpu.VMEM((B, F), jnp.float32),
            pltpu.VMEM((L,), jnp.int32),
            pltpu.VMEM((ACC_W,), jnp.float32),
            pltpu.SemaphoreType.DMA,
            pltpu.SemaphoreType.DMA,
            pltpu.SemaphoreType.DMA,
            pltpu.SemaphoreType.DMA,
            pltpu.SemaphoreType.DMA,
            pltpu.SemaphoreType.DMA,
        ],
    )


_spmm_128 = _make_spmm(D_IN, 2)
_spmm_512 = _make_spmm(H, 1)

NSL_D = D_IN // F
NSL_H = H // F


_BLK = 512
_GRID = NP // _BLK


def _pre_body(dp_ref, x_ref, dinv_ref, y0_ref):
    deg = jnp.sum(dp_ref[...], axis=1, keepdims=True) + 1.0
    dinv = lax.rsqrt(deg)
    dinv_ref[...] = dinv
    y0_ref[0, :, :] = x_ref[...] * dinv


def _k_pre(deg2, x_pad):
    return pl.pallas_call(
        _pre_body,
        grid=(_GRID,),
        in_specs=[
            pl.BlockSpec((_BLK, 2), lambda i: (i, 0)),
            pl.BlockSpec((_BLK, D_IN), lambda i: (i, 0)),
        ],
        out_specs=[
            pl.BlockSpec((_BLK, 1), lambda i: (i, 0)),
            pl.BlockSpec((NSL_D, _BLK, F), lambda i: (0, i, 0)),
        ],
        out_shape=[
            jax.ShapeDtypeStruct((NP, 1), jnp.float32),
            jax.ShapeDtypeStruct((NSL_D, NP, F), jnp.float32),
        ],
    )(deg2, x_pad)


def _layer0_body(sp_ref, y0_ref, dinv_ref, w_ref, b_ref, y1_ref):
    dinv = dinv_ref[...]
    t = (sp_ref[0, 0] + sp_ref[1, 0] + y0_ref[0]) * dinv
    h = jnp.dot(t, w_ref[...], preferred_element_type=jnp.float32) + b_ref[...]
    h = jnp.maximum(h, 0.0) * dinv
    for sl in range(NSL_H):
        y1_ref[sl, :, :] = h[:, sl * F:(sl + 1) * F]


def _k_layer0(S0p, y0, dinv, W0, b0):
    return pl.pallas_call(
        _layer0_body,
        grid=(_GRID,),
        in_specs=[
            pl.BlockSpec((2, NSL_D, _BLK, F), lambda i: (0, 0, i, 0)),
            pl.BlockSpec((NSL_D, _BLK, F), lambda i: (0, i, 0)),
            pl.BlockSpec((_BLK, 1), lambda i: (i, 0)),
            pl.BlockSpec((D_IN, H), lambda i: (0, 0)),
            pl.BlockSpec((1, H), lambda i: (0, 0)),
        ],
        out_specs=pl.BlockSpec((NSL_H, _BLK, F), lambda i: (0, i, 0)),
        out_shape=jax.ShapeDtypeStruct((NSL_H, NP, F), jnp.float32),
    )(S0p, y0, dinv, W0, b0)


def _layer1_body(s_ref, yp_ref, dinv_ref, w_ref, b_ref, y_ref):
    dinv = dinv_ref[...]
    h = b_ref[...]
    for sl in range(NSL_H):
        t = (s_ref[sl] + yp_ref[sl]) * dinv
        h = h + jnp.dot(t, w_ref[pl.ds(sl * F, F), :],
                        preferred_element_type=jnp.float32)
    h = jnp.maximum(h, 0.0) * dinv
    for sl in range(NSL_H):
        y_ref[sl, :, :] = h[:, sl * F:(sl + 1) * F]


def _k_layer1(S, yp, dinv, W, b):
    return pl.pallas_call(
        _layer1_body,
        grid=(_GRID,),
        in_specs=[
            pl.BlockSpec((NSL_H, _BLK, F), lambda i: (0, i, 0)),
            pl.BlockSpec((NSL_H, _BLK, F), lambda i: (0, i, 0)),
            pl.BlockSpec((_BLK, 1), lambda i: (i, 0)),
            pl.BlockSpec((H, H), lambda i: (0, 0)),
            pl.BlockSpec((1, H), lambda i: (0, 0)),
        ],
        out_specs=pl.BlockSpec((NSL_H, _BLK, F), lambda i: (0, i, 0)),
        out_shape=jax.ShapeDtypeStruct((NSL_H, NP, F), jnp.float32),
    )(S, yp, dinv, W, b)


def _final_body(s_ref, yp_ref, dinv_ref, w2_ref, b2_ref, wfc_ref, bfc_ref,
                o_ref):
    dinv = dinv_ref[...]
    h = b2_ref[...]
    for sl in range(NSL_H):
        t = (s_ref[sl] + yp_ref[sl]) * dinv
        h = h + jnp.dot(t, w2_ref[pl.ds(sl * F, F), :],
                        preferred_element_type=jnp.float32)
    h = jnp.maximum(h, 0.0)
    o_ref[...] = jnp.dot(
        h, wfc_ref[...], preferred_element_type=jnp.float32) + bfc_ref[...]


def _k_final(S, yp, dinv, W2, b2, Wfc, bfc):
    return pl.pallas_call(
        _final_body,
        grid=(_GRID,),
        in_specs=[
            pl.BlockSpec((NSL_H, _BLK, F), lambda i: (0, i, 0)),
            pl.BlockSpec((NSL_H, _BLK, F), lambda i: (0, i, 0)),
            pl.BlockSpec((_BLK, 1), lambda i: (i, 0)),
            pl.BlockSpec((H, H), lambda i: (0, 0)),
            pl.BlockSpec((1, H), lambda i: (0, 0)),
            pl.BlockSpec((H, D_IN), lambda i: (0, 0)),
            pl.BlockSpec((1, D_IN), lambda i: (0, 0)),
        ],
        out_specs=pl.BlockSpec((_BLK, D_IN), lambda i: (i, 0)),
        out_shape=jax.ShapeDtypeStruct((NP, D_IN), jnp.float32),
    )(S, yp, dinv, W2, b2, Wfc, bfc)


def kernel(x, edge_index, W0, b0, W1, b1, W2, b2, Wfc, bfc):
    ei_flat = edge_index.reshape(-1).astype(jnp.int32)
    x_pad = jnp.pad(x, ((0, NP - N), (0, 0)))

    bsrc, bdst, counts, degp = _preprocess(ei_flat)
    ssrc, sdst, scnt = _merge(bsrc, bdst, counts)
    deg2 = degp.reshape(NC, NP).T

    dinv, y0 = _k_pre(deg2, x_pad)
    S0 = _spmm_128(y0.reshape(NSL_D * NP, F), ssrc, sdst, scnt)
    y1 = _k_layer0(S0.reshape(2, NSL_D, NP, F), y0, dinv, W0,
                   b0.reshape(1, H))
    S1 = _spmm_512(y1.reshape(NSL_H * NP, F), ssrc, sdst, scnt)
    y2 = _k_layer1(S1.reshape(NSL_H, NP, F), y1, dinv, W1, b1.reshape(1, H))
    S2 = _spmm_512(y2.reshape(NSL_H * NP, F), ssrc, sdst, scnt)
    out = _k_final(S2.reshape(NSL_H, NP, F), y2, dinv, W2, b2.reshape(1, H),
                   Wfc, bfc.reshape(1, D_IN))
    return out[:N]
